# baseline (speedup 1.0000x reference)
import functools

import jax
import jax.numpy as jnp
from jax import lax
from jax.experimental import pallas as pl
from jax.experimental.pallas import tpu as pltpu
from jax.experimental.pallas import tpu_sc as plsc

NC = 2
NS = 16
NW = NC * NS
IDXW = 128
K = 8
NBUF = 2


def _sc_gather(idx2d, table):
    n_rows, d = table.shape
    b = idx2d.shape[0] * idx2d.shape[1]
    chunk = K * IDXW
    per_w = b // NW
    steps = per_w // chunk
    assert per_w % chunk == 0 and steps % 2 == 0 and steps >= 4

    mesh = plsc.VectorSubcoreMesh(core_axis_name="c", subcore_axis_name="s")

    @functools.partial(
        pl.kernel,
        mesh=mesh,
        out_type=jax.ShapeDtypeStruct((b, 128), jnp.float32),
        scratch_types=[
            pltpu.VMEM((NBUF, K, IDXW), jnp.int32),
            pltpu.VMEM((NBUF, chunk, d), jnp.float32),
            [pltpu.SemaphoreType.DMA] * NBUF,
            [pltpu.SemaphoreType.DMA] * NBUF,
            [pltpu.SemaphoreType.DMA] * NBUF,
        ],
        compiler_params=pltpu.CompilerParams(use_tc_tiling_on_sc=False),
    )
    def gather_kernel(table_hbm, idx_hbm, out_hbm, idx_v, rows_v,
                      sem_idx, sem_g, sem_st):
        wid = lax.axis_index("s") * NC + lax.axis_index("c")
        row0 = wid * (per_w // IDXW)
        out0 = wid * per_w

        def idx_copy(s, bb):
            sc = jnp.minimum(s, steps - 1)
            return pltpu.make_async_copy(
                idx_hbm.at[pl.ds(row0 + sc * K, K)], idx_v.at[bb], sem_idx[bb])

        def fire_gathers(bb):
            copies = []
            for j in range(K):
                copies.append(
                    pltpu.async_copy(
                        table_hbm.at[idx_v.at[bb].at[j]],
                        rows_v.at[bb].at[pl.ds(j * IDXW, IDXW)],
                        sem_g[bb],
                    ))
            return copies

        def store_copy(s, bb):
            return pltpu.make_async_copy(
                rows_v.at[bb],
                out_hbm.at[pl.ds(out0 + s * chunk, chunk), pl.ds(0, d)],
                sem_st[bb])

        def run_step(s, bb, wait_store):
            idx_copy(s, bb).wait()
            if wait_store:
                store_copy(s, bb).wait()
            for c in fire_gathers(bb):
                c.wait()
            idx_copy(s + NBUF, bb).start()
            store_copy(s, bb).start()

        for bb in range(NBUF):
            idx_copy(bb, bb).start()
        for bb in range(NBUF):
            run_step(bb, bb, wait_store=False)

        def pair(g, carry):
            s0 = NBUF + g * NBUF
            for bb in range(NBUF):
                run_step(s0 + bb, bb, wait_store=True)
            return carry

        lax.fori_loop(0, (steps - NBUF) // NBUF, pair, 0)

        for bb in range(NBUF):
            store_copy(steps - NBUF + bb, bb).wait()
            idx_copy(0, bb).wait()

    return gather_kernel(table, idx2d)


def kernel(coeffs, table):
    batch, seq = coeffs.shape
    d = table.shape[1]
    b = batch * seq
    idx2d = coeffs.reshape(b // IDXW, IDXW)
    half = idx2d.shape[0] // 2
    parts = []
    for s in range(2):
        p128 = _sc_gather(idx2d[s * half:(s + 1) * half], table)
        parts.append(p128[:, :d].reshape(batch // 2, seq, d))
    return jnp.concatenate(parts, axis=0)


# speedup vs baseline: 1.2027x; 1.2027x over previous
import functools

import jax
import jax.numpy as jnp
from jax import lax
from jax.experimental import pallas as pl
from jax.experimental.pallas import tpu as pltpu
from jax.experimental.pallas import tpu_sc as plsc

NC = 2
NS = 16
NW = NC * NS
IDXW = 128
K = 8
NBUF = 2


def _sc_gather(idx2d, table):
    n_rows, d = table.shape
    b = idx2d.shape[0] * idx2d.shape[1]
    chunk = K * IDXW
    per_w = b // NW
    steps = per_w // chunk
    assert per_w % chunk == 0 and steps % 2 == 0 and steps >= 4

    mesh = plsc.VectorSubcoreMesh(core_axis_name="c", subcore_axis_name="s")

    @functools.partial(
        pl.kernel,
        mesh=mesh,
        out_type=jax.ShapeDtypeStruct((b, 128), jnp.float32),
        scratch_types=[
            pltpu.VMEM((NBUF, K, IDXW), jnp.int32),
            pltpu.VMEM((NBUF, chunk, d), jnp.float32),
            [pltpu.SemaphoreType.DMA] * NBUF,
            [pltpu.SemaphoreType.DMA] * NBUF,
            [pltpu.SemaphoreType.DMA] * NBUF,
        ],
        compiler_params=pltpu.CompilerParams(use_tc_tiling_on_sc=False),
    )
    def gather_kernel(table_hbm, idx_hbm, out_hbm, idx_v, rows_v,
                      sem_idx, sem_g, sem_st):
        wid = lax.axis_index("s") * NC + lax.axis_index("c")
        row0 = wid * (per_w // IDXW)
        out0 = wid * per_w

        def idx_copy(s, bb):
            sc = jnp.minimum(s, steps - 1)
            return pltpu.make_async_copy(
                idx_hbm.at[pl.ds(row0 + sc * K, K)], idx_v.at[bb], sem_idx[bb])

        def fire_gathers(bb):
            copies = []
            for j in range(K):
                copies.append(
                    pltpu.async_copy(
                        table_hbm.at[idx_v.at[bb].at[j]],
                        rows_v.at[bb].at[pl.ds(j * IDXW, IDXW)],
                        sem_g[bb],
                    ))
            return copies

        def store_copy(s, bb):
            return pltpu.make_async_copy(
                rows_v.at[bb],
                out_hbm.at[pl.ds(out0 + s * chunk, chunk), pl.ds(0, d)],
                sem_st[bb])

        def run_step(s, bb, wait_store):
            idx_copy(s, bb).wait()
            if wait_store:
                store_copy(s, bb).wait()
            for c in fire_gathers(bb):
                c.wait()
            idx_copy(s + NBUF, bb).start()
            store_copy(s, bb).start()

        for bb in range(NBUF):
            idx_copy(bb, bb).start()
        for bb in range(NBUF):
            run_step(bb, bb, wait_store=False)

        def pair(g, carry):
            s0 = NBUF + g * NBUF
            for bb in range(NBUF):
                run_step(s0 + bb, bb, wait_store=True)
            return carry

        lax.fori_loop(0, (steps - NBUF) // NBUF, pair, 0)

        for bb in range(NBUF):
            store_copy(steps - NBUF + bb, bb).wait()
            idx_copy(0, bb).wait()

    return gather_kernel(table, idx2d)


def kernel(coeffs, table):
    batch, seq = coeffs.shape
    d = table.shape[1]
    b = batch * seq
    idx2d = coeffs.reshape(b // IDXW, IDXW)
    out128 = _sc_gather(idx2d, table)
    return out128[:, :d].reshape(batch, seq, d)


# speedup vs baseline: 1.2056x; 1.0024x over previous
import functools

import jax
import jax.numpy as jnp
from jax import lax
from jax.experimental import pallas as pl
from jax.experimental.pallas import tpu as pltpu
from jax.experimental.pallas import tpu_sc as plsc

NC = 2
NS = 16
NW = NC * NS
IDXW = 128
K = 10
NBUF = 2


def _sc_gather(idx2d, table):
    n_rows, d = table.shape
    b = idx2d.shape[0] * idx2d.shape[1]
    chunk = K * IDXW
    per_w = b // NW
    steps = per_w // chunk
    assert per_w % chunk == 0 and steps % 2 == 0 and steps >= 4

    mesh = plsc.VectorSubcoreMesh(core_axis_name="c", subcore_axis_name="s")

    @functools.partial(
        pl.kernel,
        mesh=mesh,
        out_type=jax.ShapeDtypeStruct((b, 128), jnp.float32),
        scratch_types=[
            pltpu.VMEM((NBUF, K, IDXW), jnp.int32),
            pltpu.VMEM((NBUF, chunk, d), jnp.float32),
            [pltpu.SemaphoreType.DMA] * NBUF,
            [pltpu.SemaphoreType.DMA] * NBUF,
            [pltpu.SemaphoreType.DMA] * NBUF,
        ],
        compiler_params=pltpu.CompilerParams(use_tc_tiling_on_sc=False),
    )
    def gather_kernel(table_hbm, idx_hbm, out_hbm, idx_v, rows_v,
                      sem_idx, sem_g, sem_st):
        wid = lax.axis_index("s") * NC + lax.axis_index("c")
        row0 = wid * (per_w // IDXW)
        out0 = wid * per_w

        def idx_copy(s, bb):
            sc = jnp.minimum(s, steps - 1)
            return pltpu.make_async_copy(
                idx_hbm.at[pl.ds(row0 + sc * K, K)], idx_v.at[bb], sem_idx[bb])

        def fire_gathers(bb):
            copies = []
            for j in range(K):
                copies.append(
                    pltpu.async_copy(
                        table_hbm.at[idx_v.at[bb].at[j]],
                        rows_v.at[bb].at[pl.ds(j * IDXW, IDXW)],
                        sem_g[bb],
                    ))
            return copies

        def store_copy(s, bb):
            return pltpu.make_async_copy(
                rows_v.at[bb],
                out_hbm.at[pl.ds(out0 + s * chunk, chunk), pl.ds(0, d)],
                sem_st[bb])

        def run_step(s, bb, wait_store):
            idx_copy(s, bb).wait()
            if wait_store:
                store_copy(s, bb).wait()
            for c in fire_gathers(bb):
                c.wait()
            idx_copy(s + NBUF, bb).start()
            store_copy(s, bb).start()

        for bb in range(NBUF):
            idx_copy(bb, bb).start()
        for bb in range(NBUF):
            run_step(bb, bb, wait_store=False)

        def pair(g, carry):
            s0 = NBUF + g * NBUF
            for bb in range(NBUF):
                run_step(s0 + bb, bb, wait_store=True)
            return carry

        lax.fori_loop(0, (steps - NBUF) // NBUF, pair, 0)

        for bb in range(NBUF):
            store_copy(steps - NBUF + bb, bb).wait()
            idx_copy(0, bb).wait()

    return gather_kernel(table, idx2d)


def kernel(coeffs, table):
    batch, seq = coeffs.shape
    d = table.shape[1]
    b = batch * seq
    idx2d = coeffs.reshape(b // IDXW, IDXW)
    out128 = _sc_gather(idx2d, table)
    return out128[:, :d].reshape(batch, seq, d)
